# Initial kernel scaffold; baseline (speedup 1.0000x reference)
import jax
import jax.numpy as jnp
from jax.experimental import pallas as pl


def kernel(p, edge_src, edge_dst):
    raise NotImplementedError("write your pallas kernel here")


# speedup vs baseline: 31.2874x; 31.2874x over previous
import functools

import jax
import jax.numpy as jnp
from jax import lax
from jax.experimental import pallas as pl
from jax.experimental.pallas import tpu as pltpu
from jax.experimental.pallas import tpu_sc as plsc

B = 16
N_NODES = 100000
N_PAD = 100096
NC = 2
NS = 16
NW = NC * NS
G = 128
GPB = 16
GROUPS_PER_WORKER = 784
NBLK = GROUPS_PER_WORKER // GPB
E_PAD = NW * GROUPS_PER_WORKER * G
ROWS_PER_TILE = N_PAD // NS


def _sc_body(pt_hbm, src_hbm, dst_hbm, zero_hbm, out_hbm,
             idx_s, idx_d, rows_s, rows_d, g_buf, ng_buf, acc,
             sem_s, sem_d):
    c = lax.axis_index("c")
    s = lax.axis_index("s")
    wid = s * NC + c

    stripe = pl.ds(s * ROWS_PER_TILE, ROWS_PER_TILE)
    pltpu.sync_copy(zero_hbm.at[stripe], acc.at[stripe])
    plsc.subcore_barrier()

    @pl.loop(0, NBLK)
    def _block(b):
        row0 = wid * GROUPS_PER_WORKER + b * GPB
        pltpu.sync_copy(src_hbm.at[pl.ds(row0, GPB)], idx_s)
        pltpu.sync_copy(dst_hbm.at[pl.ds(row0, GPB)], idx_d)

        @pl.loop(0, GPB)
        def _group(j):
            cp_s = pltpu.async_copy(pt_hbm.at[idx_s.at[j]], rows_s, sem_s)
            cp_d = pltpu.async_copy(pt_hbm.at[idx_d.at[j]], rows_d, sem_d)
            cp_s.wait()
            cp_d.wait()
            for r in range(G):
                a = rows_s[r]
                bb = rows_d[r]
                g_buf[r] = a - bb
                ng_buf[r] = bb - a
            pltpu.sync_copy(g_buf, acc.at[idx_s.at[j]], add=True)
            pltpu.sync_copy(ng_buf, acc.at[idx_d.at[j]], add=True)

    plsc.subcore_barrier()
    pltpu.sync_copy(acc.at[stripe], out_hbm.at[c, stripe])


@jax.jit
def _laplacian_sc(pt, src2d, dst2d, zero):
    mesh = plsc.VectorSubcoreMesh(
        core_axis_name="c", subcore_axis_name="s",
        num_cores=NC, num_subcores=NS)
    f = functools.partial(
        pl.kernel,
        out_type=jax.ShapeDtypeStruct((NC, N_PAD, B), jnp.float32),
        mesh=mesh,
        scratch_types=[
            pltpu.VMEM((GPB, G), jnp.int32),
            pltpu.VMEM((GPB, G), jnp.int32),
            pltpu.VMEM((G, B), jnp.float32),
            pltpu.VMEM((G, B), jnp.float32),
            pltpu.VMEM((G, B), jnp.float32),
            pltpu.VMEM((G, B), jnp.float32),
            pltpu.VMEM_SHARED((N_PAD, B), jnp.float32),
            pltpu.SemaphoreType.DMA,
            pltpu.SemaphoreType.DMA,
        ],
        compiler_params=pltpu.CompilerParams(use_tc_tiling_on_sc=False),
    )(_sc_body)
    return f(pt, src2d, dst2d, zero)


def kernel(p, edge_src, edge_dst):
    n_edges = edge_src.shape[0]
    pad = E_PAD - n_edges
    pt = jnp.pad(p.T, ((0, N_PAD - N_NODES), (0, 0)))
    src2d = jnp.concatenate(
        [edge_src, jnp.zeros((pad,), edge_src.dtype)]).reshape(-1, G)
    dst2d = jnp.concatenate(
        [edge_dst, jnp.zeros((pad,), edge_dst.dtype)]).reshape(-1, G)
    zero = jnp.zeros((N_PAD, B), jnp.float32)
    parts = _laplacian_sc(pt, src2d, dst2d, zero)
    return (parts[0, :N_NODES] + parts[1, :N_NODES]).T


# speedup vs baseline: 54.4391x; 1.7400x over previous
import functools

import jax
import jax.numpy as jnp
from jax import lax
from jax.experimental import pallas as pl
from jax.experimental.pallas import tpu as pltpu
from jax.experimental.pallas import tpu_sc as plsc

B = 16
N_NODES = 100000
N_PAD = 100096
NC = 2
NS = 16
NW = NC * NS
G = 128
GPB = 28
GROUPS_PER_WORKER = 784
NBLK = GROUPS_PER_WORKER // GPB
E_PAD = NW * GROUPS_PER_WORKER * G
ROWS_PER_TILE = N_PAD // NS


def _sc_body(pt_hbm, src_hbm, dst_hbm, zero_hbm, out_hbm,
             idx_s, idx_d, rs0, rd0, rs1, rd1, g0, ng0, g1, ng1, acc,
             sem_is, sem_id, gs0, gd0, gs1, gd1, ss0, sd0, ss1, sd1):
    c = lax.axis_index("c")
    s = lax.axis_index("s")
    wid = s * NC + c
    row_base = wid * GROUPS_PER_WORKER

    stripe = pl.ds(s * ROWS_PER_TILE, ROWS_PER_TILE)
    pltpu.sync_copy(zero_hbm.at[stripe], acc.at[stripe])

    pltpu.sync_copy(src_hbm.at[pl.ds(row_base, GPB)], idx_s.at[0])
    pltpu.sync_copy(dst_hbm.at[pl.ds(row_base, GPB)], idx_d.at[0])
    plsc.subcore_barrier()

    def compute(rs, rd, g, ng):
        for r in range(G):
            a = rs[r]
            d = rd[r]
            g[r] = a - d
            ng[r] = d - a

    @pl.loop(0, NBLK)
    def _block(b):
        par = lax.rem(b, 2)

        @pl.when(b > 0)
        def _():
            pltpu.make_async_copy(
                src_hbm.at[pl.ds(row_base + b * GPB, GPB)],
                idx_s.at[par], sem_is).wait()
            pltpu.make_async_copy(
                dst_hbm.at[pl.ds(row_base + b * GPB, GPB)],
                idx_d.at[par], sem_id).wait()

        @pl.when(b + 1 < NBLK)
        def _():
            nb = b + 1
            pltpu.async_copy(src_hbm.at[pl.ds(row_base + nb * GPB, GPB)],
                             idx_s.at[1 - par], sem_is)
            pltpu.async_copy(dst_hbm.at[pl.ds(row_base + nb * GPB, GPB)],
                             idx_d.at[1 - par], sem_id)

        pltpu.async_copy(pt_hbm.at[idx_s.at[par, 0]], rs0, gs0)
        pltpu.async_copy(pt_hbm.at[idx_d.at[par, 0]], rd0, gd0)

        @pl.loop(0, GPB // 2)
        def _pair(k):
            j0 = 2 * k
            j1 = j0 + 1

            @pl.when(k > 0)
            def _():
                pltpu.make_async_copy(
                    g0, acc.at[idx_s.at[par, j0 - 2]], ss0).wait()
                pltpu.make_async_copy(
                    ng0, acc.at[idx_d.at[par, j0 - 2]], sd0).wait()
            pltpu.async_copy(pt_hbm.at[idx_s.at[par, j1]], rs1, gs1)
            pltpu.async_copy(pt_hbm.at[idx_d.at[par, j1]], rd1, gd1)
            pltpu.make_async_copy(pt_hbm.at[idx_s.at[par, j0]], rs0, gs0).wait()
            pltpu.make_async_copy(pt_hbm.at[idx_d.at[par, j0]], rd0, gd0).wait()
            compute(rs0, rd0, g0, ng0)
            pltpu.async_copy(g0, acc.at[idx_s.at[par, j0]], ss0, add=True)
            pltpu.async_copy(ng0, acc.at[idx_d.at[par, j0]], sd0, add=True)

            @pl.when(k > 0)
            def _():
                pltpu.make_async_copy(
                    g1, acc.at[idx_s.at[par, j1 - 2]], ss1).wait()
                pltpu.make_async_copy(
                    ng1, acc.at[idx_d.at[par, j1 - 2]], sd1).wait()

            @pl.when(j0 + 2 < GPB)
            def _():
                pltpu.async_copy(pt_hbm.at[idx_s.at[par, j0 + 2]], rs0, gs0)
                pltpu.async_copy(pt_hbm.at[idx_d.at[par, j0 + 2]], rd0, gd0)
            pltpu.make_async_copy(pt_hbm.at[idx_s.at[par, j1]], rs1, gs1).wait()
            pltpu.make_async_copy(pt_hbm.at[idx_d.at[par, j1]], rd1, gd1).wait()
            compute(rs1, rd1, g1, ng1)
            pltpu.async_copy(g1, acc.at[idx_s.at[par, j1]], ss1, add=True)
            pltpu.async_copy(ng1, acc.at[idx_d.at[par, j1]], sd1, add=True)

        pltpu.make_async_copy(g0, acc.at[idx_s.at[par, GPB - 2]], ss0).wait()
        pltpu.make_async_copy(ng0, acc.at[idx_d.at[par, GPB - 2]], sd0).wait()
        pltpu.make_async_copy(g1, acc.at[idx_s.at[par, GPB - 1]], ss1).wait()
        pltpu.make_async_copy(ng1, acc.at[idx_d.at[par, GPB - 1]], sd1).wait()

    plsc.subcore_barrier()
    pltpu.sync_copy(acc.at[stripe], out_hbm.at[c, stripe])


@jax.jit
def _laplacian_sc(pt, src2d, dst2d, zero):
    mesh = plsc.VectorSubcoreMesh(
        core_axis_name="c", subcore_axis_name="s",
        num_cores=NC, num_subcores=NS)
    f = functools.partial(
        pl.kernel,
        out_type=jax.ShapeDtypeStruct((NC, N_PAD, B), jnp.float32),
        mesh=mesh,
        scratch_types=[
            pltpu.VMEM((2, GPB, G), jnp.int32),
            pltpu.VMEM((2, GPB, G), jnp.int32),
            pltpu.VMEM((G, B), jnp.float32),
            pltpu.VMEM((G, B), jnp.float32),
            pltpu.VMEM((G, B), jnp.float32),
            pltpu.VMEM((G, B), jnp.float32),
            pltpu.VMEM((G, B), jnp.float32),
            pltpu.VMEM((G, B), jnp.float32),
            pltpu.VMEM((G, B), jnp.float32),
            pltpu.VMEM((G, B), jnp.float32),
            pltpu.VMEM_SHARED((N_PAD, B), jnp.float32),
        ] + [pltpu.SemaphoreType.DMA] * 10,
        compiler_params=pltpu.CompilerParams(use_tc_tiling_on_sc=False),
    )(_sc_body)
    return f(pt, src2d, dst2d, zero)


def kernel(p, edge_src, edge_dst):
    n_edges = edge_src.shape[0]
    pad = E_PAD - n_edges
    pt = jnp.pad(p.T, ((0, N_PAD - N_NODES), (0, 0)))
    src2d = jnp.concatenate(
        [edge_src, jnp.zeros((pad,), edge_src.dtype)]).reshape(-1, G)
    dst2d = jnp.concatenate(
        [edge_dst, jnp.zeros((pad,), edge_dst.dtype)]).reshape(-1, G)
    zero = jnp.zeros((N_PAD, B), jnp.float32)
    parts = _laplacian_sc(pt, src2d, dst2d, zero)
    return (parts[0, :N_NODES] + parts[1, :N_NODES]).T
